# baseline (speedup 1.0000x reference)
import functools

import jax
import jax.numpy as jnp
from jax import lax
from jax.experimental import pallas as pl
from jax.experimental.pallas import tpu as pltpu
from jax.experimental.pallas import tpu_sc as plsc

_F = 26
_V = 100000
_D = 32
_B = 4096
_H1 = 256
_H2 = 128
_INP = _F * _D

_NC = 2
_NS = 16
_NW = _NC * _NS
_LANES = 16


def _sc_gather_xt(embT, idsT):
    mesh = plsc.VectorSubcoreMesh(core_axis_name="c", subcore_axis_name="s")

    @functools.partial(
        pl.kernel,
        out_type=jax.ShapeDtypeStruct((_F, _D, _B), jnp.float32),
        mesh=mesh,
        scratch_types=[
            pltpu.VMEM((_V,), jnp.float32),
            pltpu.VMEM((_B,), jnp.int32),
            pltpu.VMEM((_B,), jnp.float32),
            pltpu.SemaphoreType.DMA,
            pltpu.SemaphoreType.DMA,
            pltpu.SemaphoreType.DMA,
        ],
        compiler_params=pltpu.CompilerParams(use_tc_tiling_on_sc=True,
                                             needs_layout_passes=False),
    )
    def k(emb_hbm, ids_hbm, out_hbm, slab_v, ids_v, row_v, sem_s, sem_i,
          sem_r):
        d = lax.axis_index("s") * _NC + lax.axis_index("c")

        pltpu.async_copy(ids_hbm.at[0], ids_v, sem_i)

        def field_body(f, carry):
            pltpu.async_copy(emb_hbm.at[f, d], slab_v, sem_s)
            pltpu.make_async_copy(ids_hbm.at[0], ids_v, sem_i).wait()

            @pl.when(f > 0)
            def _wait_prev_row():
                pltpu.make_async_copy(row_v, out_hbm.at[0, 0], sem_r).wait()

            pltpu.make_async_copy(emb_hbm.at[f, d], slab_v, sem_s).wait()

            def gather_body(i, c2):
                s = pl.ds(pl.multiple_of(i * _LANES, _LANES), _LANES)
                row_v[s] = plsc.load_gather(slab_v, [ids_v[s]])
                return c2

            lax.fori_loop(0, _B // _LANES, gather_body, 0)
            pltpu.async_copy(row_v, out_hbm.at[f, d], sem_r)

            @pl.when(f < _F - 1)
            def _prefetch_ids():
                pltpu.async_copy(ids_hbm.at[f + 1], ids_v, sem_i)

            return carry

        lax.fori_loop(0, _F, field_body, 0)
        pltpu.make_async_copy(row_v, out_hbm.at[0, 0], sem_r).wait()

    return k(embT, idsT)


_BLK = 512


def _mlp_body(x_ref, cw1, cb1, cw2, cb2, cw3, cb3, vw1, vb1, vw2, vb2, vw3,
              vb3, out_ref):
    x = x_ref[...].astype(jnp.bfloat16)

    def tower(w1, b1, w2, b2, w3):
        h = jnp.maximum(
            lax.dot_general(w1[...].astype(jnp.bfloat16), x,
                            (((0,), (0,)), ((), ())),
                            preferred_element_type=jnp.float32) + b1[...], 0.0)
        h = h.astype(jnp.bfloat16)
        h = jnp.maximum(
            lax.dot_general(w2[...].astype(jnp.bfloat16), h,
                            (((0,), (0,)), ((), ())),
                            preferred_element_type=jnp.float32) + b2[...], 0.0)
        h = h.astype(jnp.bfloat16)
        return lax.dot_general(w3[...].astype(jnp.bfloat16), h,
                               (((0,), (0,)), ((), ())),
                               preferred_element_type=jnp.float32)

    lc = tower(cw1, cb1, cw2, cb2, cw3) + cb3[0, 0]
    lv = tower(vw1, vb1, vw2, vb2, vw3) + vb3[0, 0]
    ctr = 1.0 / (1.0 + jnp.exp(-lc))
    cvr = 1.0 / (1.0 + jnp.exp(-lv))
    out_ref[...] = jnp.concatenate([ctr, cvr, ctr * cvr], axis=0)


def _mlp(xT, cW1p, cb1, cW2, cb2, cW3, cb3, vW1p, vb1, vW2, vb2, vW3, vb3):
    wspec = lambda shape: pl.BlockSpec(shape, lambda i: (0, 0))
    return pl.pallas_call(
        _mlp_body,
        grid=(_B // _BLK,),
        in_specs=[
            pl.BlockSpec((_INP, _BLK), lambda i: (0, i)),
            wspec((_INP, _H1)),
            wspec((_H1, 1)),
            wspec((_H1, _H2)),
            wspec((_H2, 1)),
            wspec((_H2, 1)),
            pl.BlockSpec(memory_space=pltpu.SMEM),
            wspec((_INP, _H1)),
            wspec((_H1, 1)),
            wspec((_H1, _H2)),
            wspec((_H2, 1)),
            wspec((_H2, 1)),
            pl.BlockSpec(memory_space=pltpu.SMEM),
        ],
        out_specs=pl.BlockSpec((3, _BLK), lambda i: (0, i)),
        out_shape=jax.ShapeDtypeStruct((3, _B), jnp.float32),
    )(xT, cW1p, cb1, cW2, cb2, cW3, cb3, vW1p, vb1, vW2, vb2, vW3, vb3)


def kernel(ids, emb_tables, cW1, cb1, cW2, cb2, cW3, cb3, vW1, vb1, vW2, vb2,
           vW3, vb3):
    embT = jnp.transpose(emb_tables, (0, 2, 1))
    idsT = ids.T
    xT = _sc_gather_xt(embT, idsT).reshape(_INP, _B)
    out = _mlp(xT, cW1, cb1.reshape(_H1, 1), cW2, cb2.reshape(_H2, 1),
               cW3.reshape(_H2, 1), cb3.reshape(1, 1), vW1,
               vb1.reshape(_H1, 1), vW2, vb2.reshape(_H2, 1),
               vW3.reshape(_H2, 1), vb3.reshape(1, 1))
    return out[0], out[1], out[2]


# speedup vs baseline: 1.1025x; 1.1025x over previous
import functools

import jax
import jax.numpy as jnp
from jax import lax
from jax.experimental import pallas as pl
from jax.experimental.pallas import tpu as pltpu
from jax.experimental.pallas import tpu_sc as plsc

_F = 26
_V = 100000
_D = 32
_B = 4096
_H1 = 256
_H2 = 128
_INP = _F * _D

_NC = 2
_NS = 16
_NW = _NC * _NS
_LANES = 16


def _sc_gather_xt(embT, idsT):
    mesh = plsc.VectorSubcoreMesh(core_axis_name="c", subcore_axis_name="s")

    @functools.partial(
        pl.kernel,
        out_type=jax.ShapeDtypeStruct((_F, _D, _B), jnp.float32),
        mesh=mesh,
        scratch_types=[
            pltpu.VMEM((_V,), jnp.float32),
            pltpu.VMEM((_B,), jnp.int32),
            pltpu.VMEM((_B,), jnp.float32),
        ],
        compiler_params=pltpu.CompilerParams(use_tc_tiling_on_sc=True,
                                             needs_layout_passes=False),
    )
    def k(emb_hbm, ids_hbm, out_hbm, slab_v, ids_v, row_v):
        d = lax.axis_index("s") * _NC + lax.axis_index("c")

        def field_body(f, carry):
            pltpu.sync_copy(ids_hbm.at[f], ids_v)
            pltpu.sync_copy(emb_hbm.at[f, d], slab_v)

            def gather_body(i, c2):
                base = pl.multiple_of(i * (4 * _LANES), _LANES)
                for j in range(4):
                    s = pl.ds(pl.multiple_of(base + j * _LANES, _LANES),
                              _LANES)
                    row_v[s] = plsc.load_gather(slab_v, [ids_v[s]])
                return c2

            lax.fori_loop(0, _B // (4 * _LANES), gather_body, 0)
            pltpu.sync_copy(row_v, out_hbm.at[f, d])
            return carry

        lax.fori_loop(0, _F, field_body, 0)

    return k(embT, idsT)


_BLK = 512


def _mlp_body(x_ref, cw1, cb1, cw2, cb2, cw3, cb3, vw1, vb1, vw2, vb2, vw3,
              vb3, out_ref):
    x = x_ref[...].astype(jnp.bfloat16)

    def tower(w1, b1, w2, b2, w3):
        h = jnp.maximum(
            lax.dot_general(w1[...].astype(jnp.bfloat16), x,
                            (((0,), (0,)), ((), ())),
                            preferred_element_type=jnp.float32) + b1[...], 0.0)
        h = h.astype(jnp.bfloat16)
        h = jnp.maximum(
            lax.dot_general(w2[...].astype(jnp.bfloat16), h,
                            (((0,), (0,)), ((), ())),
                            preferred_element_type=jnp.float32) + b2[...], 0.0)
        h = h.astype(jnp.bfloat16)
        return lax.dot_general(w3[...].astype(jnp.bfloat16), h,
                               (((0,), (0,)), ((), ())),
                               preferred_element_type=jnp.float32)

    lc = tower(cw1, cb1, cw2, cb2, cw3) + cb3[0, 0]
    lv = tower(vw1, vb1, vw2, vb2, vw3) + vb3[0, 0]
    ctr = 1.0 / (1.0 + jnp.exp(-lc))
    cvr = 1.0 / (1.0 + jnp.exp(-lv))
    out_ref[...] = jnp.concatenate([ctr, cvr, ctr * cvr], axis=0)


def _mlp(xT, cW1p, cb1, cW2, cb2, cW3, cb3, vW1p, vb1, vW2, vb2, vW3, vb3):
    wspec = lambda shape: pl.BlockSpec(shape, lambda i: (0, 0))
    return pl.pallas_call(
        _mlp_body,
        grid=(_B // _BLK,),
        in_specs=[
            pl.BlockSpec((_INP, _BLK), lambda i: (0, i)),
            wspec((_INP, _H1)),
            wspec((_H1, 1)),
            wspec((_H1, _H2)),
            wspec((_H2, 1)),
            wspec((_H2, 1)),
            pl.BlockSpec(memory_space=pltpu.SMEM),
            wspec((_INP, _H1)),
            wspec((_H1, 1)),
            wspec((_H1, _H2)),
            wspec((_H2, 1)),
            wspec((_H2, 1)),
            pl.BlockSpec(memory_space=pltpu.SMEM),
        ],
        out_specs=pl.BlockSpec((3, _BLK), lambda i: (0, i)),
        out_shape=jax.ShapeDtypeStruct((3, _B), jnp.float32),
    )(xT, cW1p, cb1, cW2, cb2, cW3, cb3, vW1p, vb1, vW2, vb2, vW3, vb3)


def kernel(ids, emb_tables, cW1, cb1, cW2, cb2, cW3, cb3, vW1, vb1, vW2, vb2,
           vW3, vb3):
    embT = jnp.transpose(emb_tables, (0, 2, 1))
    idsT = ids.T
    xT = _sc_gather_xt(embT, idsT).reshape(_INP, _B)
    out = _mlp(xT, cW1, cb1.reshape(_H1, 1), cW2, cb2.reshape(_H2, 1),
               cW3.reshape(_H2, 1), cb3.reshape(1, 1), vW1,
               vb1.reshape(_H1, 1), vW2, vb2.reshape(_H2, 1),
               vW3.reshape(_H2, 1), vb3.reshape(1, 1))
    return out[0], out[1], out[2]


# speedup vs baseline: 1.1032x; 1.0007x over previous
import functools

import jax
import jax.numpy as jnp
from jax import lax
from jax.experimental import pallas as pl
from jax.experimental.pallas import tpu as pltpu
from jax.experimental.pallas import tpu_sc as plsc

_F = 26
_V = 100000
_D = 32
_B = 4096
_H1 = 256
_H2 = 128
_INP = _F * _D

_NC = 2
_NS = 16
_NW = _NC * _NS
_LANES = 16


def _sc_gather_xt(embT, idsT):
    mesh = plsc.VectorSubcoreMesh(core_axis_name="c", subcore_axis_name="s")

    @functools.partial(
        pl.kernel,
        out_type=jax.ShapeDtypeStruct((_F, _D, _B), jnp.float32),
        mesh=mesh,
        scratch_types=[
            pltpu.VMEM((_V,), jnp.float32),
            pltpu.VMEM((_B,), jnp.int32),
            pltpu.VMEM((_B,), jnp.float32),
        ],
        compiler_params=pltpu.CompilerParams(use_tc_tiling_on_sc=True,
                                             needs_layout_passes=False),
    )
    def k(emb_hbm, ids_hbm, out_hbm, slab_v, ids_v, row_v):
        d = lax.axis_index("s") * _NC + lax.axis_index("c")

        def field_body(f, carry):
            pltpu.sync_copy(ids_hbm.at[f], ids_v)
            pltpu.sync_copy(emb_hbm.at[f, d], slab_v)

            def gather_body(i, c2):
                base = pl.multiple_of(i * (8 * _LANES), _LANES)
                for j in range(8):
                    s = pl.ds(pl.multiple_of(base + j * _LANES, _LANES),
                              _LANES)
                    row_v[s] = plsc.load_gather(slab_v, [ids_v[s]])
                return c2

            lax.fori_loop(0, _B // (8 * _LANES), gather_body, 0)
            pltpu.sync_copy(row_v, out_hbm.at[f, d])
            return carry

        lax.fori_loop(0, _F, field_body, 0)

    return k(embT, idsT)


_BLK = 512


def _mlp_body(x_ref, cw1, cb1, cw2, cb2, cw3, cb3, vw1, vb1, vw2, vb2, vw3,
              vb3, out_ref):
    x = x_ref[...].astype(jnp.bfloat16)

    def tower(w1, b1, w2, b2, w3):
        h = jnp.maximum(
            lax.dot_general(w1[...].astype(jnp.bfloat16), x,
                            (((0,), (0,)), ((), ())),
                            preferred_element_type=jnp.float32) + b1[...], 0.0)
        h = h.astype(jnp.bfloat16)
        h = jnp.maximum(
            lax.dot_general(w2[...].astype(jnp.bfloat16), h,
                            (((0,), (0,)), ((), ())),
                            preferred_element_type=jnp.float32) + b2[...], 0.0)
        h = h.astype(jnp.bfloat16)
        return lax.dot_general(w3[...].astype(jnp.bfloat16), h,
                               (((0,), (0,)), ((), ())),
                               preferred_element_type=jnp.float32)

    lc = tower(cw1, cb1, cw2, cb2, cw3) + cb3[0, 0]
    lv = tower(vw1, vb1, vw2, vb2, vw3) + vb3[0, 0]
    ctr = 1.0 / (1.0 + jnp.exp(-lc))
    cvr = 1.0 / (1.0 + jnp.exp(-lv))
    out_ref[...] = jnp.concatenate([ctr, cvr, ctr * cvr], axis=0)


def _mlp(xT, cW1p, cb1, cW2, cb2, cW3, cb3, vW1p, vb1, vW2, vb2, vW3, vb3):
    wspec = lambda shape: pl.BlockSpec(shape, lambda i: (0, 0))
    return pl.pallas_call(
        _mlp_body,
        grid=(_B // _BLK,),
        in_specs=[
            pl.BlockSpec((_INP, _BLK), lambda i: (0, i)),
            wspec((_INP, _H1)),
            wspec((_H1, 1)),
            wspec((_H1, _H2)),
            wspec((_H2, 1)),
            wspec((_H2, 1)),
            pl.BlockSpec(memory_space=pltpu.SMEM),
            wspec((_INP, _H1)),
            wspec((_H1, 1)),
            wspec((_H1, _H2)),
            wspec((_H2, 1)),
            wspec((_H2, 1)),
            pl.BlockSpec(memory_space=pltpu.SMEM),
        ],
        out_specs=pl.BlockSpec((3, _BLK), lambda i: (0, i)),
        out_shape=jax.ShapeDtypeStruct((3, _B), jnp.float32),
    )(xT, cW1p, cb1, cW2, cb2, cW3, cb3, vW1p, vb1, vW2, vb2, vW3, vb3)


def kernel(ids, emb_tables, cW1, cb1, cW2, cb2, cW3, cb3, vW1, vb1, vW2, vb2,
           vW3, vb3):
    embT = jnp.transpose(emb_tables, (0, 2, 1))
    idsT = ids.T
    xT = _sc_gather_xt(embT, idsT).reshape(_INP, _B)
    out = _mlp(xT, cW1, cb1.reshape(_H1, 1), cW2, cb2.reshape(_H2, 1),
               cW3.reshape(_H2, 1), cb3.reshape(1, 1), vW1,
               vb1.reshape(_H1, 1), vW2, vb2.reshape(_H2, 1),
               vW3.reshape(_H2, 1), vb3.reshape(1, 1))
    return out[0], out[1], out[2]


# speedup vs baseline: 1.1238x; 1.0187x over previous
import functools

import jax
import jax.numpy as jnp
from jax import lax
from jax.experimental import pallas as pl
from jax.experimental.pallas import tpu as pltpu
from jax.experimental.pallas import tpu_sc as plsc

_F = 26
_V = 100000
_D = 32
_B = 4096
_H1 = 256
_H2 = 128
_INP = _F * _D

_NC = 2
_NS = 16
_NW = _NC * _NS
_LANES = 16


def _sc_gather_xt(embT, idsT):
    mesh = plsc.VectorSubcoreMesh(core_axis_name="c", subcore_axis_name="s")

    @functools.partial(
        pl.kernel,
        out_type=jax.ShapeDtypeStruct((_F, _D, _B), jnp.float32),
        mesh=mesh,
        scratch_types=[
            pltpu.VMEM((_V,), jnp.float32),
            pltpu.VMEM((_B,), jnp.int32),
            pltpu.VMEM((_B,), jnp.float32),
        ],
        compiler_params=pltpu.CompilerParams(use_tc_tiling_on_sc=True,
                                             needs_layout_passes=False),
    )
    def k(emb_hbm, ids_hbm, out_hbm, slab_v, ids_v, row_v):
        d = lax.axis_index("s") * _NC + lax.axis_index("c")

        def field_body(f, carry):
            pltpu.sync_copy(ids_hbm.at[f], ids_v)
            pltpu.sync_copy(emb_hbm.at[f, d], slab_v)

            def gather_body(i, c2):
                base = pl.multiple_of(i * (8 * _LANES), _LANES)
                for j in range(8):
                    s = pl.ds(pl.multiple_of(base + j * _LANES, _LANES),
                              _LANES)
                    row_v[s] = plsc.load_gather(slab_v, [ids_v[s]])
                return c2

            lax.fori_loop(0, _B // (8 * _LANES), gather_body, 0)
            pltpu.sync_copy(row_v, out_hbm.at[f, d])
            return carry

        lax.fori_loop(0, _F, field_body, 0)

    return k(embT, idsT)


_BLK = 1024


def _mlp_body(x_ref, cw1, cb1, cw2, cb2, cw3, cb3, vw1, vb1, vw2, vb2, vw3,
              vb3, out_ref):
    x = x_ref[...].astype(jnp.bfloat16)

    def tower(w1, b1, w2, b2, w3):
        h = jnp.maximum(
            lax.dot_general(w1[...].astype(jnp.bfloat16), x,
                            (((0,), (0,)), ((), ())),
                            preferred_element_type=jnp.float32) + b1[...], 0.0)
        h = h.astype(jnp.bfloat16)
        h = jnp.maximum(
            lax.dot_general(w2[...].astype(jnp.bfloat16), h,
                            (((0,), (0,)), ((), ())),
                            preferred_element_type=jnp.float32) + b2[...], 0.0)
        h = h.astype(jnp.bfloat16)
        return lax.dot_general(w3[...].astype(jnp.bfloat16), h,
                               (((0,), (0,)), ((), ())),
                               preferred_element_type=jnp.float32)

    lc = tower(cw1, cb1, cw2, cb2, cw3) + cb3[0, 0]
    lv = tower(vw1, vb1, vw2, vb2, vw3) + vb3[0, 0]
    ctr = 1.0 / (1.0 + jnp.exp(-lc))
    cvr = 1.0 / (1.0 + jnp.exp(-lv))
    out_ref[...] = jnp.concatenate([ctr, cvr, ctr * cvr], axis=0)


def _mlp(xT, cW1p, cb1, cW2, cb2, cW3, cb3, vW1p, vb1, vW2, vb2, vW3, vb3):
    wspec = lambda shape: pl.BlockSpec(shape, lambda i: (0, 0))
    return pl.pallas_call(
        _mlp_body,
        grid=(_B // _BLK,),
        in_specs=[
            pl.BlockSpec((_INP, _BLK), lambda i: (0, i)),
            wspec((_INP, _H1)),
            wspec((_H1, 1)),
            wspec((_H1, _H2)),
            wspec((_H2, 1)),
            wspec((_H2, 1)),
            pl.BlockSpec(memory_space=pltpu.SMEM),
            wspec((_INP, _H1)),
            wspec((_H1, 1)),
            wspec((_H1, _H2)),
            wspec((_H2, 1)),
            wspec((_H2, 1)),
            pl.BlockSpec(memory_space=pltpu.SMEM),
        ],
        out_specs=pl.BlockSpec((3, _BLK), lambda i: (0, i)),
        out_shape=jax.ShapeDtypeStruct((3, _B), jnp.float32),
    )(xT, cW1p, cb1, cW2, cb2, cW3, cb3, vW1p, vb1, vW2, vb2, vW3, vb3)


def kernel(ids, emb_tables, cW1, cb1, cW2, cb2, cW3, cb3, vW1, vb1, vW2, vb2,
           vW3, vb3):
    embT = jnp.transpose(emb_tables, (0, 2, 1))
    idsT = ids.T
    xT = _sc_gather_xt(embT, idsT).reshape(_INP, _B)
    out = _mlp(xT, cW1, cb1.reshape(_H1, 1), cW2, cb2.reshape(_H2, 1),
               cW3.reshape(_H2, 1), cb3.reshape(1, 1), vW1,
               vb1.reshape(_H1, 1), vW2, vb2.reshape(_H2, 1),
               vW3.reshape(_H2, 1), vb3.reshape(1, 1))
    return out[0], out[1], out[2]


# speedup vs baseline: 1.1290x; 1.0046x over previous
import functools

import jax
import jax.numpy as jnp
from jax import lax
from jax.experimental import pallas as pl
from jax.experimental.pallas import tpu as pltpu
from jax.experimental.pallas import tpu_sc as plsc

_F = 26
_V = 100000
_D = 32
_B = 4096
_H1 = 256
_H2 = 128
_INP = _F * _D

_NC = 2
_NS = 16
_NW = _NC * _NS
_LANES = 16


def _sc_gather_xt(embT, idsT):
    mesh = plsc.VectorSubcoreMesh(core_axis_name="c", subcore_axis_name="s")

    @functools.partial(
        pl.kernel,
        out_type=jax.ShapeDtypeStruct((_F, _D, _B), jnp.float32),
        mesh=mesh,
        scratch_types=[
            pltpu.VMEM((_V,), jnp.float32),
            pltpu.VMEM((_B,), jnp.int32),
            pltpu.VMEM((_B,), jnp.float32),
        ],
        compiler_params=pltpu.CompilerParams(use_tc_tiling_on_sc=True,
                                             needs_layout_passes=False),
    )
    def k(emb_hbm, ids_hbm, out_hbm, slab_v, ids_v, row_v):
        d = lax.axis_index("s") * _NC + lax.axis_index("c")

        def field_body(f, carry):
            pltpu.sync_copy(ids_hbm.at[f], ids_v)
            pltpu.sync_copy(emb_hbm.at[f, d], slab_v)

            def gather_body(i, c2):
                base = pl.multiple_of(i * (8 * _LANES), _LANES)
                for j in range(8):
                    s = pl.ds(pl.multiple_of(base + j * _LANES, _LANES),
                              _LANES)
                    row_v[s] = plsc.load_gather(slab_v, [ids_v[s]])
                return c2

            lax.fori_loop(0, _B // (8 * _LANES), gather_body, 0)
            pltpu.sync_copy(row_v, out_hbm.at[f, d])
            return carry

        lax.fori_loop(0, _F, field_body, 0)

    return k(embT, idsT)


_BLK = 2048


def _mlp_body(x_ref, cw1, cb1, cw2, cb2, cw3, cb3, vw1, vb1, vw2, vb2, vw3,
              vb3, out_ref):
    x = x_ref[...].astype(jnp.bfloat16)

    def tower(w1, b1, w2, b2, w3):
        h = jnp.maximum(
            lax.dot_general(w1[...].astype(jnp.bfloat16), x,
                            (((0,), (0,)), ((), ())),
                            preferred_element_type=jnp.float32) + b1[...], 0.0)
        h = h.astype(jnp.bfloat16)
        h = jnp.maximum(
            lax.dot_general(w2[...].astype(jnp.bfloat16), h,
                            (((0,), (0,)), ((), ())),
                            preferred_element_type=jnp.float32) + b2[...], 0.0)
        h = h.astype(jnp.bfloat16)
        return lax.dot_general(w3[...].astype(jnp.bfloat16), h,
                               (((0,), (0,)), ((), ())),
                               preferred_element_type=jnp.float32)

    lc = tower(cw1, cb1, cw2, cb2, cw3) + cb3[0, 0]
    lv = tower(vw1, vb1, vw2, vb2, vw3) + vb3[0, 0]
    ctr = 1.0 / (1.0 + jnp.exp(-lc))
    cvr = 1.0 / (1.0 + jnp.exp(-lv))
    out_ref[...] = jnp.concatenate([ctr, cvr, ctr * cvr], axis=0)


def _mlp(xT, cW1p, cb1, cW2, cb2, cW3, cb3, vW1p, vb1, vW2, vb2, vW3, vb3):
    wspec = lambda shape: pl.BlockSpec(shape, lambda i: (0, 0))
    return pl.pallas_call(
        _mlp_body,
        grid=(_B // _BLK,),
        in_specs=[
            pl.BlockSpec((_INP, _BLK), lambda i: (0, i)),
            wspec((_INP, _H1)),
            wspec((_H1, 1)),
            wspec((_H1, _H2)),
            wspec((_H2, 1)),
            wspec((_H2, 1)),
            pl.BlockSpec(memory_space=pltpu.SMEM),
            wspec((_INP, _H1)),
            wspec((_H1, 1)),
            wspec((_H1, _H2)),
            wspec((_H2, 1)),
            wspec((_H2, 1)),
            pl.BlockSpec(memory_space=pltpu.SMEM),
        ],
        out_specs=pl.BlockSpec((3, _BLK), lambda i: (0, i)),
        out_shape=jax.ShapeDtypeStruct((3, _B), jnp.float32),
    )(xT, cW1p, cb1, cW2, cb2, cW3, cb3, vW1p, vb1, vW2, vb2, vW3, vb3)


def kernel(ids, emb_tables, cW1, cb1, cW2, cb2, cW3, cb3, vW1, vb1, vW2, vb2,
           vW3, vb3):
    embT = jnp.transpose(emb_tables, (0, 2, 1))
    idsT = ids.T
    xT = _sc_gather_xt(embT, idsT).reshape(_INP, _B)
    out = _mlp(xT, cW1, cb1.reshape(_H1, 1), cW2, cb2.reshape(_H2, 1),
               cW3.reshape(_H2, 1), cb3.reshape(1, 1), vW1,
               vb1.reshape(_H1, 1), vW2, vb2.reshape(_H2, 1),
               vW3.reshape(_H2, 1), vb3.reshape(1, 1))
    return out[0], out[1], out[2]
